# baseline (speedup 1.0000x reference)
import functools

import jax
import jax.numpy as jnp
from jax import lax
from jax.experimental import pallas as pl
from jax.experimental.pallas import tpu as pltpu
from jax.experimental.pallas import tpu_sc as plsc

_EMBED_DIM = 32
_GROUP = 640
_NBUF = 4


def _make_lookup(num_idx: int):
    info = plsc.get_sparse_core_info()
    n_cores, n_sub = info.num_cores, info.num_subcores
    n_workers = n_cores * n_sub
    per_w = num_idx // n_workers
    n_groups = per_w // _GROUP

    mesh = plsc.VectorSubcoreMesh(core_axis_name="c", subcore_axis_name="s")

    scratch = [pltpu.VMEM((per_w,), jnp.int32)]
    scratch += [pltpu.VMEM((_GROUP, _EMBED_DIM), jnp.float32)] * _NBUF
    scratch += [pltpu.SemaphoreType.DMA] * (2 * _NBUF)

    @functools.partial(
        pl.kernel,
        mesh=mesh,
        out_type=jax.ShapeDtypeStruct((num_idx, _EMBED_DIM), jnp.float32),
        scratch_types=scratch,
        compiler_params=pltpu.CompilerParams(use_tc_tiling_on_sc=False),
    )
    def lookup(idx_hbm, table_hbm, out_hbm, idx_v, *rest):
        bufs = rest[:_NBUF]
        gsems = rest[_NBUF:2 * _NBUF]
        wsems = rest[2 * _NBUF:]
        wid = lax.axis_index("s") * n_cores + lax.axis_index("c")
        base = wid * per_w

        def gather(g, b):
            pltpu.async_copy(
                table_hbm.at[idx_v.at[pl.ds(g * _GROUP, _GROUP)]],
                bufs[b], gsems[b],
            )

        def drain_gather(b):
            pltpu.make_async_copy(
                table_hbm.at[pl.ds(0, _GROUP)], bufs[b], gsems[b]
            ).wait()

        def writeback(g, b):
            pltpu.async_copy(
                bufs[b], out_hbm.at[pl.ds(base + g * _GROUP, _GROUP)], wsems[b]
            )

        def drain_wb(b):
            pltpu.make_async_copy(
                bufs[b], out_hbm.at[pl.ds(base, _GROUP)], wsems[b]
            ).wait()

        pltpu.sync_copy(idx_hbm.at[pl.ds(base, per_w)], idx_v)
        for b in range(_NBUF):
            gather(b, b)

        def body(h, carry):
            g = _NBUF * h
            for b in range(_NBUF):
                drain_gather(b)
                writeback(g + b, b)
            for b in range(_NBUF):
                drain_wb(b)
                gather(g + _NBUF + b, b)
            return carry

        lax.fori_loop(0, n_groups // _NBUF - 1, body, 0)

        g_last = n_groups - _NBUF
        for b in range(_NBUF):
            drain_gather(b)
            writeback(g_last + b, b)
        for b in range(_NBUF):
            drain_wb(b)

    return lookup


def kernel(token_ids, weight):
    s0, s1 = token_ids.shape
    num_idx = s0 * s1
    idx = token_ids.reshape(num_idx).astype(jnp.int32)
    out = _make_lookup(num_idx)(idx, weight)
    return out.reshape(s0, s1, _EMBED_DIM)
